# baseline (speedup 1.0000x reference)
import functools

import jax
import jax.numpy as jnp
from jax import lax
from jax.experimental import pallas as pl
from jax.experimental.pallas import tpu as pltpu
from jax.experimental.pallas import tpu_sc as plsc

NC = 2
NS = 16
L = 16

RG = 4


def _nnd_sc_body(x_hbm, y_hbm, out1, out2,
                 x_ref, y_ref, cm_ref, rm_ref, tmp_ref, ob_ref, shared,
                 *, b_per_c, rows_w, m_tot):
    c = lax.axis_index("c")
    s = lax.axis_index("s")
    inf16 = jnp.full((L,), jnp.inf, jnp.float32)
    iota = lax.iota(jnp.int32, L)
    mchunks = m_tot // L

    for bl in range(b_per_c):
        b = c * b_per_c + bl
        pltpu.sync_copy(y_hbm.at[b], y_ref)
        pltpu.sync_copy(x_hbm.at[b, s], x_ref)

        def init_body(i, carry):
            cm_ref[pl.ds(i * L, L)] = inf16
            return carry
        lax.fori_loop(0, m_tot // L, init_body, 0)

        def group_body(g, carry):
            r0 = g * RG
            bc = [[x_ref[pl.ds(((d * rows_w) + r0 + r) * L, L)]
                   for d in range(3)] for r in range(RG)]

            def sweep(i, rms):
                off = i * L
                y0 = y_ref[pl.ds(off, L)]
                y1 = y_ref[pl.ds(m_tot + off, L)]
                y2 = y_ref[pl.ds(2 * m_tot + off, L)]
                cm = cm_ref[pl.ds(off, L)]
                out = []
                for r in range(RG):
                    d0 = y0 - bc[r][0]
                    t = d0 * d0
                    d1 = y1 - bc[r][1]
                    t = t + d1 * d1
                    d2 = y2 - bc[r][2]
                    t = t + d2 * d2
                    out.append(jnp.minimum(rms[r], t))
                    cm = jnp.minimum(cm, t)
                cm_ref[pl.ds(off, L)] = cm
                return tuple(out)

            rms = lax.fori_loop(0, mchunks, sweep, (inf16,) * RG,
                                unroll=2)

            chunk = (r0 // L) * L
            rv = rm_ref[pl.ds(chunk, L)]
            base_lane = r0 % L
            for r in range(RG):
                mn = rms[r]
                for sh in (8, 4, 2, 1):
                    idx = (iota + sh) & (L - 1)
                    rot = lax.gather(
                        mn, idx[:, None],
                        lax.GatherDimensionNumbers(
                            offset_dims=(), collapsed_slice_dims=(0,),
                            start_index_map=(0,)),
                        slice_sizes=(1,),
                        mode=lax.GatherScatterMode.PROMISE_IN_BOUNDS)
                    mn = jnp.minimum(mn, rot)
                rv = jnp.where(iota == base_lane + r, mn, rv)
            rm_ref[pl.ds(chunk, L)] = rv
            return carry

        lax.fori_loop(0, rows_w // RG, group_body, 0)

        pltpu.sync_copy(rm_ref, out1.at[b, pl.ds(s * rows_w, rows_w)])

        pltpu.sync_copy(cm_ref, shared.at[s])
        plsc.subcore_barrier()
        cols_w = m_tot // NS
        pltpu.sync_copy(shared.at[:, pl.ds(s * cols_w, cols_w)], tmp_ref)
        plsc.subcore_barrier()

        def red_body(j, carry):
            acc = tmp_ref[0, pl.ds(j * L, L)]
            for i in range(1, NS):
                acc = jnp.minimum(acc, tmp_ref[i, pl.ds(j * L, L)])
            ob_ref[pl.ds(j * L, L)] = acc
            return carry
        lax.fori_loop(0, cols_w // L, red_body, 0)

        pltpu.sync_copy(ob_ref, out2.at[b, pl.ds(s * cols_w, cols_w)])


@jax.jit
def kernel(input1, input2):
    b, n, _ = input1.shape
    m = input2.shape[1]
    rows_w = n // NS
    b_per_c = b // NC

    xt = input1.transpose(0, 2, 1)
    xg = xt.reshape(b, 3, NS, rows_w).transpose(0, 2, 1, 3)
    xw = jnp.broadcast_to(xg[..., None], (b, NS, 3, rows_w, L))
    xw = xw.reshape(b, NS, 3 * rows_w * L)
    yf = input2.transpose(0, 2, 1).reshape(b, 3 * m)

    mesh = plsc.VectorSubcoreMesh(core_axis_name="c", subcore_axis_name="s",
                                  num_cores=NC, num_subcores=NS)
    body = functools.partial(_nnd_sc_body, b_per_c=b_per_c,
                             rows_w=rows_w, m_tot=m)
    d1, d2 = pl.kernel(
        body,
        out_type=[jax.ShapeDtypeStruct((b, n), jnp.float32),
                  jax.ShapeDtypeStruct((b, m), jnp.float32)],
        mesh=mesh,
        scratch_types=[
            pltpu.VMEM((3 * rows_w * L,), jnp.float32),
            pltpu.VMEM((3 * m,), jnp.float32),
            pltpu.VMEM((m,), jnp.float32),
            pltpu.VMEM((rows_w,), jnp.float32),
            pltpu.VMEM((NS, m // NS), jnp.float32),
            pltpu.VMEM((m // NS,), jnp.float32),
            pltpu.VMEM_SHARED((NS, m), jnp.float32),
        ],
    )(xw, yf)
    return d1, d2


# speedup vs baseline: 1.2490x; 1.2490x over previous
import functools

import jax
import jax.numpy as jnp
from jax import lax
from jax.experimental import pallas as pl
from jax.experimental.pallas import tpu as pltpu
from jax.experimental.pallas import tpu_sc as plsc

NC = 2
NS = 16
L = 16

RG = 8


def _nnd_sc_body(x_hbm, y_hbm, out1, out2,
                 x_ref, y_ref, cm_ref, rm_ref, tmp_ref, ob_ref, shared,
                 *, b_per_c, rows_w, m_tot):
    c = lax.axis_index("c")
    s = lax.axis_index("s")
    inf16 = jnp.full((L,), jnp.inf, jnp.float32)
    iota = lax.iota(jnp.int32, L)
    mchunks = m_tot // L

    for bl in range(b_per_c):
        b = c * b_per_c + bl
        pltpu.sync_copy(y_hbm.at[b], y_ref)
        pltpu.sync_copy(x_hbm.at[b, s], x_ref)

        def init_body(i, carry):
            cm_ref[pl.ds(i * L, L)] = inf16
            return carry
        lax.fori_loop(0, m_tot // L, init_body, 0)

        def group_body(g, carry):
            r0 = g * RG
            bc = [[x_ref[pl.ds(((d * rows_w) + r0 + r) * L, L)]
                   for d in range(3)] for r in range(RG)]

            @plsc.parallel_loop(0, mchunks, carry=(inf16,) * RG, unroll=2)
            def rms(i, rms_c):
                off = i * L
                y0 = y_ref[pl.ds(off, L)]
                y1 = y_ref[pl.ds(m_tot + off, L)]
                y2 = y_ref[pl.ds(2 * m_tot + off, L)]
                cm = cm_ref[pl.ds(off, L)]
                out = []
                for r in range(RG):
                    d0 = y0 - bc[r][0]
                    t = d0 * d0
                    d1 = y1 - bc[r][1]
                    t = t + d1 * d1
                    d2 = y2 - bc[r][2]
                    t = t + d2 * d2
                    out.append(jnp.minimum(rms_c[r], t))
                    cm = jnp.minimum(cm, t)
                cm_ref[pl.ds(off, L)] = cm
                return tuple(out)

            chunk = (r0 // L) * L
            rv = rm_ref[pl.ds(chunk, L)]
            base_lane = r0 % L
            for r in range(RG):
                mn = rms[r]
                for sh in (8, 4, 2, 1):
                    idx = (iota + sh) & (L - 1)
                    rot = lax.gather(
                        mn, idx[:, None],
                        lax.GatherDimensionNumbers(
                            offset_dims=(), collapsed_slice_dims=(0,),
                            start_index_map=(0,)),
                        slice_sizes=(1,),
                        mode=lax.GatherScatterMode.PROMISE_IN_BOUNDS)
                    mn = jnp.minimum(mn, rot)
                rv = jnp.where(iota == base_lane + r, mn, rv)
            rm_ref[pl.ds(chunk, L)] = rv
            return carry

        lax.fori_loop(0, rows_w // RG, group_body, 0)

        pltpu.sync_copy(rm_ref, out1.at[b, pl.ds(s * rows_w, rows_w)])

        pltpu.sync_copy(cm_ref, shared.at[s])
        plsc.subcore_barrier()
        cols_w = m_tot // NS
        pltpu.sync_copy(shared.at[:, pl.ds(s * cols_w, cols_w)], tmp_ref)
        plsc.subcore_barrier()

        def red_body(j, carry):
            acc = tmp_ref[0, pl.ds(j * L, L)]
            for i in range(1, NS):
                acc = jnp.minimum(acc, tmp_ref[i, pl.ds(j * L, L)])
            ob_ref[pl.ds(j * L, L)] = acc
            return carry
        lax.fori_loop(0, cols_w // L, red_body, 0)

        pltpu.sync_copy(ob_ref, out2.at[b, pl.ds(s * cols_w, cols_w)])


@jax.jit
def kernel(input1, input2):
    b, n, _ = input1.shape
    m = input2.shape[1]
    rows_w = n // NS
    b_per_c = b // NC

    xt = input1.transpose(0, 2, 1)
    xg = xt.reshape(b, 3, NS, rows_w).transpose(0, 2, 1, 3)
    xw = jnp.broadcast_to(xg[..., None], (b, NS, 3, rows_w, L))
    xw = xw.reshape(b, NS, 3 * rows_w * L)
    yf = input2.transpose(0, 2, 1).reshape(b, 3 * m)

    mesh = plsc.VectorSubcoreMesh(core_axis_name="c", subcore_axis_name="s",
                                  num_cores=NC, num_subcores=NS)
    body = functools.partial(_nnd_sc_body, b_per_c=b_per_c,
                             rows_w=rows_w, m_tot=m)
    d1, d2 = pl.kernel(
        body,
        out_type=[jax.ShapeDtypeStruct((b, n), jnp.float32),
                  jax.ShapeDtypeStruct((b, m), jnp.float32)],
        mesh=mesh,
        scratch_types=[
            pltpu.VMEM((3 * rows_w * L,), jnp.float32),
            pltpu.VMEM((3 * m,), jnp.float32),
            pltpu.VMEM((m,), jnp.float32),
            pltpu.VMEM((rows_w,), jnp.float32),
            pltpu.VMEM((NS, m // NS), jnp.float32),
            pltpu.VMEM((m // NS,), jnp.float32),
            pltpu.VMEM_SHARED((NS, m), jnp.float32),
        ],
    )(xw, yf)
    return d1, d2
